# baseline (speedup 1.0000x reference)
import jax
import jax.numpy as jnp
from jax.experimental import pallas as pl
from jax.experimental.pallas import tpu as pltpu

_HW = 1024
_HC = 512
_NCLS = 150
_C = 96
_TAU = 0.3
_BB = 8


def _body(prob_ref, emb_ref, out_ref, mstate, sstate, oacc):
    h = pl.program_id(1)
    for i in range(_BB):
        p = prob_ref[i]
        e = emb_ref[i]
        mC = jnp.max(p, axis=0, keepdims=True)
        ones_row = jnp.ones((1, _HC), jnp.float32)
        sC = jax.lax.dot_general(ones_row, p, (((1,), (0,)), ((), ())),
                                 preferred_element_type=jnp.float32)
        hw_iota = jax.lax.broadcasted_iota(jnp.int32, p.shape, 0)
        iC = jnp.min(jnp.where(p == mC, hw_iota, _HC), axis=0,
                     keepdims=True)
        oh = (hw_iota == iC).astype(jnp.float32)
        cand = jax.lax.dot_general(
            oh, e, (((0,), (0,)), ((), ())),
            preferred_element_type=jnp.float32)

        @pl.when(h == 0)
        def _():
            mstate[i] = mC
            sstate[i] = sC
            oacc[i] = cand

        @pl.when(h == 1)
        def _():
            r_io = jax.lax.broadcasted_iota(jnp.int32, (_NCLS, _NCLS), 0)
            c_io = jax.lax.broadcasted_iota(jnp.int32, (_NCLS, _NCLS), 1)
            eye = (r_io == c_io).astype(jnp.float32)
            upd = (mC > mstate[i]).astype(jnp.float32)
            s = sstate[i] + sC
            rep = ((s * (1.0 / _HW)) > _TAU).astype(jnp.float32)
            upd_t = jax.lax.dot_general(
                eye, upd, (((1,), (1,)), ((), ())),
                preferred_element_type=jnp.float32)
            rep_t = jax.lax.dot_general(
                eye, rep, (((1,), (1,)), ((), ())),
                preferred_element_type=jnp.float32)
            merged = cand * upd_t + oacc[i] * (1.0 - upd_t)
            out_ref[i] = merged * rep_t


def kernel(emb, prob_map):
    B = emb.shape[0]
    emb_flat = emb.reshape(B, _HW, _C)
    prob_flat = prob_map.reshape(B, _HW, _NCLS)
    out = pl.pallas_call(
        _body,
        grid=(B // _BB, _HW // _HC),
        in_specs=[
            pl.BlockSpec((_BB, _HC, _NCLS), lambda b, h: (b, h, 0)),
            pl.BlockSpec((_BB, _HC, _C), lambda b, h: (b, h, 0)),
        ],
        out_specs=pl.BlockSpec((_BB, _NCLS, _C), lambda b, h: (b, 0, 0)),
        out_shape=jax.ShapeDtypeStruct((B, _NCLS, _C), jnp.float32),
        scratch_shapes=[
            pltpu.VMEM((_BB, 1, _NCLS), jnp.float32),
            pltpu.VMEM((_BB, 1, _NCLS), jnp.float32),
            pltpu.VMEM((_BB, _NCLS, _C), jnp.float32),
        ],
    )(prob_flat, emb_flat)
    return out


# speedup vs baseline: 1.5861x; 1.5861x over previous
import jax
import jax.numpy as jnp
from jax.experimental import pallas as pl

_H, _W, _C = 32, 32, 96
_HW = _H * _W
_NCLS = 150
_TAU = 0.3


def _body(prob_ref, emb_ref, out_ref):
  for i in range(8):
    p = prob_ref[i]
    e = emb_ref[i]
    m = jnp.max(p, axis=0, keepdims=True)
    ones_row = jnp.ones((1, _HW), jnp.float32)
    s = jax.lax.dot_general(ones_row, p, (((1,), (0,)), ((), ())),
                            preferred_element_type=jnp.float32)
    hw_iota = jax.lax.broadcasted_iota(jnp.int32, p.shape, 0)
    idx = jnp.min(jnp.where(p == m, hw_iota, _HW), axis=0, keepdims=True)
    rep = (s * (1.0 / _HW)) > _TAU
    onehot = ((hw_iota == idx) & rep).astype(jnp.float32)
    out_ref[i] = jax.lax.dot_general(
        onehot, e, (((0,), (0,)), ((), ())),
        preferred_element_type=jnp.float32,
    )


def kernel(emb, prob_map):
    B = emb.shape[0]
    emb_flat = emb.reshape(B, _HW, _C)
    prob_flat = prob_map.reshape(B, _HW, _NCLS)
    out = pl.pallas_call(
        _body,
        grid=(B // 8,),
        in_specs=[
            pl.BlockSpec((8, _HW, _NCLS), lambda b: (b, 0, 0)),
            pl.BlockSpec((8, _HW, _C), lambda b: (b, 0, 0)),
        ],
        out_specs=pl.BlockSpec((8, _NCLS, _C), lambda b: (b, 0, 0)),
        out_shape=jax.ShapeDtypeStruct((B, _NCLS, _C), jnp.float32),
    )(prob_flat, emb_flat)
    return out
